# Initial kernel scaffold; baseline (speedup 1.0000x reference)
import jax
import jax.numpy as jnp
from jax.experimental import pallas as pl


def kernel(x, emb_weight):
    raise NotImplementedError("write your pallas kernel here")


# speedup vs baseline: 1.5842x; 1.5842x over previous
import functools

import jax
import jax.numpy as jnp
from jax import lax
from jax.experimental import pallas as pl
from jax.experimental.pallas import tpu as pltpu
from jax.experimental.pallas import tpu_sc as plsc


def _make_sc_row_copy(rows: int, dim: int):
    info = plsc.get_sparse_core_info()
    num_cores, num_subcores = info.num_cores, info.num_subcores
    num_workers = num_cores * num_subcores
    rows_per_worker = rows // num_workers
    chunk = 32
    while rows_per_worker % chunk:
        chunk //= 2
    n_chunks = rows_per_worker // chunk

    mesh = plsc.VectorSubcoreMesh(core_axis_name="c", subcore_axis_name="s")

    @functools.partial(
        pl.kernel,
        out_type=jax.ShapeDtypeStruct((rows, dim), jnp.float32),
        mesh=mesh,
        scratch_types=[
            pltpu.VMEM((chunk, dim), jnp.float32),
            pltpu.VMEM((chunk, dim), jnp.float32),
            pltpu.SemaphoreType.DMA,
            pltpu.SemaphoreType.DMA,
            pltpu.SemaphoreType.DMA,
            pltpu.SemaphoreType.DMA,
        ],
    )
    def copy_kernel(table, out, buf0, buf1, r0, r1, w0, w1):
        wid = lax.axis_index("s") * num_cores + lax.axis_index("c")
        base = wid * rows_per_worker
        bufs = (buf0, buf1)
        rsems = (r0, r1)
        wsems = (w0, w1)
        reads = [None, None]
        writes = [None, None]
        reads[0] = pltpu.make_async_copy(table.at[pl.ds(base, chunk)], buf0, r0)
        reads[0].start()
        for i in range(n_chunks):
            b = i & 1
            nb = b ^ 1
            if i + 1 < n_chunks:
                if writes[nb] is not None:
                    writes[nb].wait()
                reads[nb] = pltpu.make_async_copy(
                    table.at[pl.ds(base + (i + 1) * chunk, chunk)],
                    bufs[nb],
                    rsems[nb],
                )
                reads[nb].start()
            reads[b].wait()
            writes[b] = pltpu.make_async_copy(
                bufs[b], out.at[pl.ds(base + i * chunk, chunk)], wsems[b]
            )
            writes[b].start()
        for w in writes:
            if w is not None:
                w.wait()

    return copy_kernel


def kernel(x, emb_weight):
    seq = x.shape[1]
    _, dim = emb_weight.shape
    out = _make_sc_row_copy(seq, dim)(emb_weight)
    return out[None]


# speedup vs baseline: 1.6208x; 1.0231x over previous
import functools

import jax
import jax.numpy as jnp
from jax import lax
from jax.experimental import pallas as pl
from jax.experimental.pallas import tpu as pltpu
from jax.experimental.pallas import tpu_sc as plsc


def _make_sc_row_copy(rows: int, dim: int, chunk: int = 32, nbuf: int = 2):
    info = plsc.get_sparse_core_info()
    num_cores, num_subcores = info.num_cores, info.num_subcores
    num_workers = num_cores * num_subcores
    rows_per_worker = rows // num_workers
    while rows_per_worker % chunk:
        chunk //= 2
    n_chunks = rows_per_worker // chunk
    nbuf = min(nbuf, n_chunks)

    mesh = plsc.VectorSubcoreMesh(core_axis_name="c", subcore_axis_name="s")

    @functools.partial(
        pl.kernel,
        out_type=jax.ShapeDtypeStruct((rows, dim), jnp.float32),
        mesh=mesh,
        scratch_types=(
            [pltpu.VMEM((chunk, dim), jnp.float32)] * nbuf
            + [pltpu.SemaphoreType.DMA] * (2 * nbuf)
        ),
    )
    def copy_kernel(table, out, *refs):
        bufs = refs[:nbuf]
        rsems = refs[nbuf : 2 * nbuf]
        wsems = refs[2 * nbuf :]
        wid = lax.axis_index("s") * num_cores + lax.axis_index("c")
        base = wid * rows_per_worker
        reads = [None] * nbuf
        writes = [None] * nbuf

        def start_read(i):
            b = i % nbuf
            reads[b] = pltpu.make_async_copy(
                table.at[pl.ds(base + i * chunk, chunk)], bufs[b], rsems[b]
            )
            reads[b].start()

        for i in range(nbuf - 1):
            start_read(i)
        for i in range(n_chunks):
            b = i % nbuf
            j = i + nbuf - 1
            if j < n_chunks:
                prev = writes[j % nbuf]
                if prev is not None:
                    prev.wait()
                start_read(j)
            reads[b].wait()
            writes[b] = pltpu.make_async_copy(
                bufs[b], out.at[pl.ds(base + i * chunk, chunk)], wsems[b]
            )
            writes[b].start()
        for i in range(max(0, n_chunks - nbuf + 1), n_chunks):
            writes[i % nbuf].wait()

    return copy_kernel


def kernel(x, emb_weight):
    seq = x.shape[1]
    _, dim = emb_weight.shape
    out = _make_sc_row_copy(seq, dim, chunk=32, nbuf=3)(emb_weight)
    return out[None]


# speedup vs baseline: 1.6251x; 1.0026x over previous
import functools

import jax
import jax.numpy as jnp
from jax import lax
from jax.experimental import pallas as pl
from jax.experimental.pallas import tpu as pltpu
from jax.experimental.pallas import tpu_sc as plsc


def _make_sc_row_copy(rows: int, dim: int, chunk: int = 32, nbuf: int = 2):
    info = plsc.get_sparse_core_info()
    num_cores, num_subcores = info.num_cores, info.num_subcores
    num_workers = num_cores * num_subcores
    rows_per_worker = rows // num_workers
    while rows_per_worker % chunk:
        chunk //= 2
    n_chunks = rows_per_worker // chunk
    nbuf = min(nbuf, n_chunks)

    mesh = plsc.VectorSubcoreMesh(core_axis_name="c", subcore_axis_name="s")

    @functools.partial(
        pl.kernel,
        out_type=jax.ShapeDtypeStruct((rows, dim), jnp.float32),
        mesh=mesh,
        scratch_types=(
            [pltpu.VMEM((chunk, dim), jnp.float32)] * nbuf
            + [pltpu.SemaphoreType.DMA] * (2 * nbuf)
        ),
    )
    def copy_kernel(table, out, *refs):
        bufs = refs[:nbuf]
        rsems = refs[nbuf : 2 * nbuf]
        wsems = refs[2 * nbuf :]
        wid = lax.axis_index("s") * num_cores + lax.axis_index("c")
        base = wid * rows_per_worker
        reads = [None] * nbuf
        writes = [None] * nbuf

        def start_read(i):
            b = i % nbuf
            reads[b] = pltpu.make_async_copy(
                table.at[pl.ds(base + i * chunk, chunk)], bufs[b], rsems[b]
            )
            reads[b].start()

        for i in range(nbuf - 1):
            start_read(i)
        for i in range(n_chunks):
            b = i % nbuf
            j = i + nbuf - 1
            if j < n_chunks:
                prev = writes[j % nbuf]
                if prev is not None:
                    prev.wait()
                start_read(j)
            reads[b].wait()
            writes[b] = pltpu.make_async_copy(
                bufs[b], out.at[pl.ds(base + i * chunk, chunk)], wsems[b]
            )
            writes[b].start()
        for i in range(max(0, n_chunks - nbuf), n_chunks):
            writes[i % nbuf].wait()

    return copy_kernel


def kernel(x, emb_weight):
    seq = x.shape[1]
    _, dim = emb_weight.shape
    out = _make_sc_row_copy(seq, dim, chunk=32, nbuf=3)(emb_weight)
    return out[None]


# speedup vs baseline: 1.6281x; 1.0019x over previous
import functools

import jax
import jax.numpy as jnp
from jax import lax
from jax.experimental import pallas as pl
from jax.experimental.pallas import tpu as pltpu
from jax.experimental.pallas import tpu_sc as plsc


def _make_sc_row_copy(rows: int, dim: int, chunk: int = 32, nbuf: int = 2):
    info = plsc.get_sparse_core_info()
    num_cores, num_subcores = info.num_cores, info.num_subcores
    num_workers = num_cores * num_subcores
    rows_per_worker = rows // num_workers
    while rows_per_worker % chunk:
        chunk //= 2
    n_chunks = rows_per_worker // chunk
    nbuf = min(nbuf, n_chunks)

    mesh = plsc.VectorSubcoreMesh(core_axis_name="c", subcore_axis_name="s")

    @functools.partial(
        pl.kernel,
        out_type=jax.ShapeDtypeStruct((rows, dim), jnp.float32),
        mesh=mesh,
        scratch_types=(
            [pltpu.VMEM((chunk, dim), jnp.float32)] * nbuf
            + [pltpu.SemaphoreType.DMA] * (2 * nbuf)
        ),
    )
    def copy_kernel(table, out, *refs):
        bufs = refs[:nbuf]
        rsems = refs[nbuf : 2 * nbuf]
        wsems = refs[2 * nbuf :]
        wid = lax.axis_index("s") * num_cores + lax.axis_index("c")
        base = wid * rows_per_worker
        reads = [None] * nbuf
        writes = [None] * nbuf

        def start_read(i):
            b = i % nbuf
            reads[b] = pltpu.make_async_copy(
                table.at[pl.ds(base + i * chunk, chunk)], bufs[b], rsems[b]
            )
            reads[b].start()

        for i in range(nbuf - 1):
            start_read(i)
        for i in range(n_chunks):
            b = i % nbuf
            j = i + nbuf - 1
            if j < n_chunks:
                prev = writes[j % nbuf]
                if prev is not None:
                    prev.wait()
                start_read(j)
            reads[b].wait()
            writes[b] = pltpu.make_async_copy(
                bufs[b], out.at[pl.ds(base + i * chunk, chunk)], wsems[b]
            )
            writes[b].start()
        for i in range(max(0, n_chunks - nbuf), n_chunks):
            writes[i % nbuf].wait()

    return copy_kernel


def kernel(x, emb_weight):
    seq = x.shape[1]
    _, dim = emb_weight.shape
    out = _make_sc_row_copy(seq, dim, chunk=16, nbuf=6)(emb_weight)
    return out[None]
